# baseline (speedup 1.0000x reference)
import jax
import jax.numpy as jnp
from jax import lax
from jax.experimental import pallas as pl
from jax.experimental.pallas import tpu as pltpu
from jax.experimental.pallas import tpu_sc as plsc

N_TOKENS = 32768
INPUT_DIM = 768
N_EXPERTS = 64
BLOCK_T = 4096

CHUNK_BLOCKS = (8,)

NC = 2
NS = 16
L = 16
NW = NC * NS
ILP = 8


def _logits_body(x_ref, w_ref, b_ref, out_ref):
    lg = lax.dot_general(
        w_ref[...], x_ref[...], (((1,), (1,)), ((), ())),
        preferred_element_type=jnp.float32) + b_ref[...]
    tiles = BLOCK_T // (N_TOKENS // NW)
    out_ref[...] = jnp.swapaxes(
        lg.reshape(N_EXPERTS, tiles, N_TOKENS // NW), 0, 1)


def _logits_t(x, W, b, blk0, nblk):
    tiles = BLOCK_T // (N_TOKENS // NW)
    tpw = N_TOKENS // NW
    return pl.pallas_call(
        _logits_body,
        grid=(nblk,),
        in_specs=[
            pl.BlockSpec((BLOCK_T, INPUT_DIM), lambda i: (blk0 + i, 0)),
            pl.BlockSpec((N_EXPERTS, INPUT_DIM), lambda i: (0, 0)),
            pl.BlockSpec((N_EXPERTS, 1), lambda i: (0, 0)),
        ],
        out_specs=pl.BlockSpec((tiles, N_EXPERTS, tpw),
                               lambda i: (i, 0, 0)),
        out_shape=jax.ShapeDtypeStruct((nblk * tiles, N_EXPERTS, tpw),
                                       jnp.float32),
    )(x, W, b.reshape(N_EXPERTS, 1))


def _make_route_body(tpw):
    ng = tpw // L

    nbuf = 4
    slab = tpw // nbuf
    slab_sgs = (ng // ILP) // nbuf

    def _route_body(lg_hbm, gate_hbm, idx_hbm, lt, gv, iv, s0, s1, s2, s3):
        sems = (s0, s1, s2, s3)
        wid = lax.axis_index("s") * NC + lax.axis_index("c")
        base = wid * tpw
        cps = [
            pltpu.async_copy(
                lg_hbm.at[wid, :, pl.ds(j * slab, slab)],
                lt.at[:, pl.ds(j * slab, slab)], sems[j])
            for j in range(nbuf)
        ]

        def super_group(sg, carry):
            for j in range(nbuf):
                @pl.when(sg == j * slab_sgs)
                def _wait(_j=j):
                    cps[_j].wait()
            neg = jnp.full((L,), -jnp.inf, jnp.float32)
            zero = jnp.zeros((L,), jnp.int32)
            v1 = [neg] * ILP
            v2 = [neg] * ILP
            i1 = [zero] * ILP
            i2 = [zero] * ILP
            for e in range(N_EXPERTS):
                ei = jnp.full((L,), e, jnp.int32)
                for k in range(ILP):
                    v = lt[e, pl.ds(sg * (ILP * L) + k * L, L)]
                    gt1 = v > v1[k]
                    gt2 = v > v2[k]
                    lo = jnp.minimum(v1[k], v)
                    i2[k] = jnp.where(gt1, i1[k],
                                      jnp.where(gt2, ei, i2[k]))
                    i1[k] = jnp.where(gt1, ei, i1[k])
                    v2[k] = jnp.maximum(v2[k], lo)
                    v1[k] = jnp.maximum(v1[k], v)
            for k in range(ILP):
                p1 = 1.0 / (1.0 + jnp.exp(v2[k] - v1[k]))
                off = sg * 256 + k * L
                gv[pl.ds(off, L)] = p1
                gv[pl.ds(off + 128, L)] = 1.0 - p1
                iv[pl.ds(off, L)] = i1[k]
                iv[pl.ds(off + 128, L)] = i2[k]
            return carry

        lax.fori_loop(0, ng // ILP, super_group, 0)
        pltpu.sync_copy(gv, gate_hbm.at[pl.ds(2 * base, 2 * tpw)])
        pltpu.sync_copy(iv, idx_hbm.at[pl.ds(2 * base, 2 * tpw)])

    return _route_body


def kernel(x, W, b):
    mesh = plsc.VectorSubcoreMesh(
        core_axis_name="c", subcore_axis_name="s",
        num_cores=NC, num_subcores=NS)
    gates, idxs = [], []
    blk0 = 0
    for nblk in CHUNK_BLOCKS:
        ct = nblk * BLOCK_T
        tpw = ct // NW
        route = pl.kernel(
            _make_route_body(tpw),
            out_type=[
                jax.ShapeDtypeStruct((2 * ct,), jnp.float32),
                jax.ShapeDtypeStruct((2 * ct,), jnp.int32),
            ],
            mesh=mesh,
            scratch_types=[
                pltpu.VMEM((N_EXPERTS, tpw), jnp.float32),
                pltpu.VMEM((2 * tpw,), jnp.float32),
                pltpu.VMEM((2 * tpw,), jnp.int32),
                pltpu.SemaphoreType.DMA,
                pltpu.SemaphoreType.DMA,
                pltpu.SemaphoreType.DMA,
                pltpu.SemaphoreType.DMA,
            ],
            compiler_params=pltpu.CompilerParams(needs_layout_passes=False),
        )
        lg = _logits_t(x, W, b, blk0, nblk)
        g, i = route(lg)
        gates.append(g)
        idxs.append(i)
        blk0 += nblk
    gate128 = jnp.concatenate(gates)
    idx128 = jnp.concatenate(idxs)
    gp = gate128.reshape(256, 2, 128).transpose(0, 2, 1).reshape(N_TOKENS, 2)
    ii = idx128.reshape(256, 2, 128).transpose(0, 2, 1).reshape(N_TOKENS, 2)
    return (gp, ii)


# speedup vs baseline: 1.0410x; 1.0410x over previous
import jax
import jax.numpy as jnp
from jax import lax
from jax.experimental import pallas as pl
from jax.experimental.pallas import tpu as pltpu
from jax.experimental.pallas import tpu_sc as plsc

N_TOKENS = 32768
INPUT_DIM = 768
N_EXPERTS = 64
BLOCK_T = 4096

NC = 2
NS = 16
L = 16
NW = NC * NS
TPW = N_TOKENS // NW
NG = TPW // L


def _logits_body(x_ref, w_ref, b_ref, out_ref):
    out_ref[...] = lax.dot_general(
        w_ref[...], x_ref[...], (((1,), (1,)), ((), ())),
        preferred_element_type=jnp.float32) + b_ref[...]


def _logits_t(x, W, b):
    grid = (N_TOKENS // BLOCK_T,)
    return pl.pallas_call(
        _logits_body,
        grid=grid,
        in_specs=[
            pl.BlockSpec((BLOCK_T, INPUT_DIM), lambda i: (i, 0)),
            pl.BlockSpec((N_EXPERTS, INPUT_DIM), lambda i: (0, 0)),
            pl.BlockSpec((N_EXPERTS, 1), lambda i: (0, 0)),
        ],
        out_specs=pl.BlockSpec((N_EXPERTS, BLOCK_T), lambda i: (0, i)),
        out_shape=jax.ShapeDtypeStruct((N_EXPERTS, N_TOKENS), jnp.float32),
    )(x, W, b.reshape(N_EXPERTS, 1))


def _route_body(lg_hbm, gate_hbm, idx_hbm, lt, gv, iv):
    wid = lax.axis_index("s") * NC + lax.axis_index("c")
    base = wid * TPW
    pltpu.sync_copy(lg_hbm.at[:, pl.ds(base, TPW)], lt)

    ilp = 8

    def super_group(sg, carry):
        neg = jnp.full((L,), -jnp.inf, jnp.float32)
        zero = jnp.zeros((L,), jnp.int32)
        v1 = [neg] * ilp
        v2 = [neg] * ilp
        i1 = [zero] * ilp
        i2 = [zero] * ilp
        for e in range(N_EXPERTS):
            ei = jnp.full((L,), e, jnp.int32)
            for k in range(ilp):
                v = lt[e, pl.ds(sg * (ilp * L) + k * L, L)]
                gt1 = v > v1[k]
                gt2 = v > v2[k]
                lo = jnp.minimum(v1[k], v)
                i2[k] = jnp.where(gt1, i1[k],
                                  jnp.where(gt2, ei, i2[k]))
                i1[k] = jnp.where(gt1, ei, i1[k])
                v2[k] = jnp.maximum(v2[k], lo)
                v1[k] = jnp.maximum(v1[k], v)
        for k in range(ilp):
            p1 = 1.0 / (1.0 + jnp.exp(v2[k] - v1[k]))
            off = sg * 256 + k * L
            gv[pl.ds(off, L)] = p1
            gv[pl.ds(off + 128, L)] = 1.0 - p1
            iv[pl.ds(off, L)] = i1[k]
            iv[pl.ds(off + 128, L)] = i2[k]
        return carry

    lax.fori_loop(0, NG // ilp, super_group, 0)
    pltpu.sync_copy(gv, gate_hbm.at[pl.ds(2 * base, 2 * TPW)])
    pltpu.sync_copy(iv, idx_hbm.at[pl.ds(2 * base, 2 * TPW)])


def kernel(x, W, b):
    lg = _logits_t(x, W, b)
    mesh = plsc.VectorSubcoreMesh(
        core_axis_name="c", subcore_axis_name="s",
        num_cores=NC, num_subcores=NS)
    route = pl.kernel(
        _route_body,
        out_type=[
            jax.ShapeDtypeStruct((2 * N_TOKENS,), jnp.float32),
            jax.ShapeDtypeStruct((2 * N_TOKENS,), jnp.int32),
        ],
        mesh=mesh,
        scratch_types=[
            pltpu.VMEM((N_EXPERTS, TPW), jnp.float32),
            pltpu.VMEM((2 * TPW,), jnp.float32),
            pltpu.VMEM((2 * TPW,), jnp.int32),
        ],
        compiler_params=pltpu.CompilerParams(needs_layout_passes=False),
    )
    gate128, idx128 = route(lg)
    gp = gate128.reshape(256, 2, 128).transpose(0, 2, 1).reshape(N_TOKENS, 2)
    ii = idx128.reshape(256, 2, 128).transpose(0, 2, 1).reshape(N_TOKENS, 2)
    return (gp, ii)
